# Initial kernel scaffold; baseline (speedup 1.0000x reference)
import jax
import jax.numpy as jnp
from jax.experimental import pallas as pl


def kernel(uu_edge_index, uu_edge_vals, ii_edge_index, ii_edge_vals, uEmbeds, iEmbeds, W, b):
    raise NotImplementedError("write your pallas kernel here")


# speedup vs baseline: 4.1914x; 4.1914x over previous
import jax
import jax.numpy as jnp
from jax import lax
from jax.experimental import pallas as pl
from jax.experimental.pallas import tpu as pltpu
from jax.experimental.pallas import tpu_sc as plsc

N_SIDE = 10000
LATDIM = 128
E_SIDE = 320000
NC, NS, L = 2, 16, 16
CHUNK = 128
CHUNKS_PER_SUB = -(-E_SIDE // (NS * CHUNK))
EDGES_PER_SUB = CHUNKS_PER_SUB * CHUNK
REC = 2 * CHUNK
EROWS = 80
NRCHUNK = N_SIDE // EROWS


def _sc_body(packed_hbm, vals_hbm, emb_hbm, out_hbm, buf, vals_c, rows_v, accum, sem):
    c = lax.axis_index("c")
    s = lax.axis_index("s")
    zeros16 = jnp.zeros((L,), jnp.float32)

    nrc = (NRCHUNK - 1 - s) // NS + 1

    def _zrow(r, carry):
        for k in range(LATDIM // L):
            rows_v[r, pl.ds(k * L, L)] = zeros16
        return carry
    lax.fori_loop(0, EROWS, _zrow, None)

    def _zchunk(t, carry):
        pltpu.sync_copy(rows_v.at[pl.ds(0, EROWS)],
                        accum.at[pl.ds((s + t * NS) * EROWS, EROWS)])
        return carry
    lax.fori_loop(0, nrc, _zchunk, None)
    plsc.subcore_barrier()

    w = c * NS + s

    def _chunk(j, carry):
        off = (w * CHUNKS_PER_SUB + j) * REC
        pltpu.sync_copy(packed_hbm.at[pl.ds(off, REC)], buf)
        voff = (w * CHUNKS_PER_SUB + j) * CHUNK
        pltpu.sync_copy(vals_hbm.at[pl.ds(voff, CHUNK)], vals_c)
        pltpu.async_copy(emb_hbm.at[buf.at[pl.ds(0, CHUNK)]],
                         rows_v.at[pl.ds(0, CHUNK)], sem).wait()

        for g in range(CHUNK // L):
            dvec = buf[pl.ds(CHUNK + g * L, L)]
            val16 = vals_c[pl.ds(g * L, L)]

            def _edge(lane, cc):
                bval = lax.gather(
                    val16, jnp.full((L, 1), lane, jnp.int32),
                    dimension_numbers=lax.GatherDimensionNumbers(
                        offset_dims=(), collapsed_slice_dims=(0,),
                        start_index_map=(0,)),
                    slice_sizes=(1,),
                    mode=lax.GatherScatterMode.PROMISE_IN_BOUNDS)
                e = g * L + lane
                for k in range(LATDIM // L):
                    sl = pl.ds(k * L, L)
                    rows_v[e, sl] = rows_v[e, sl] * bval
                return cc
            lax.fori_loop(0, L, _edge, None)

            pltpu.sync_copy(rows_v.at[pl.ds(g * L, L)], accum.at[dvec],
                            add=True)
        return carry
    lax.fori_loop(0, CHUNKS_PER_SUB, _chunk, None)
    plsc.subcore_barrier()

    def _echunk(t, carry):
        r0 = (s + t * NS) * EROWS
        pltpu.sync_copy(accum.at[pl.ds(r0, EROWS)], rows_v.at[pl.ds(0, EROWS)])
        pltpu.sync_copy(emb_hbm.at[pl.ds(c * N_SIDE + r0, EROWS)],
                        rows_v.at[pl.ds(EROWS, EROWS)])

        def _erow(r, cc):
            for k in range(LATDIM // L):
                sl = pl.ds(k * L, L)
                rows_v[r, sl] = jnp.maximum(rows_v[r, sl], 0.0) + \
                    rows_v[EROWS + r, sl]
            return cc
        lax.fori_loop(0, EROWS, _erow, None)
        pltpu.sync_copy(rows_v.at[pl.ds(0, EROWS)],
                        out_hbm.at[pl.ds(c * N_SIDE + r0, EROWS)])
        return carry
    lax.fori_loop(0, nrc, _echunk, None)


_sc_spmm = pl.kernel(
    _sc_body,
    out_type=jax.ShapeDtypeStruct((2 * N_SIDE, LATDIM), jnp.float32),
    mesh=plsc.VectorSubcoreMesh(core_axis_name="c", subcore_axis_name="s",
                                num_cores=NC, num_subcores=NS),
    scratch_types=[
        pltpu.VMEM((REC,), jnp.int32),
        pltpu.VMEM((CHUNK,), jnp.float32),
        pltpu.VMEM((2 * EROWS, LATDIM), jnp.float32),
        pltpu.VMEM_SHARED((N_SIDE, LATDIM), jnp.float32),
        pltpu.SemaphoreType.DMA,
    ],
)


def _tc_dense_body(h_ref, w_ref, b_ref, o_ref):
    acc = jnp.dot(h_ref[...], w_ref[...], preferred_element_type=jnp.float32)
    o_ref[...] = jnp.maximum(2.0 * acc + b_ref[...], 0.0)


def _tc_dense(h, W, b2):
    blk = 1000
    n = h.shape[0] // blk
    return pl.pallas_call(
        _tc_dense_body,
        grid=(n,),
        in_specs=[
            pl.BlockSpec((blk, LATDIM), lambda i: (i, 0)),
            pl.BlockSpec((LATDIM, LATDIM), lambda i: (0, 0)),
            pl.BlockSpec((1, LATDIM), lambda i: (0, 0)),
        ],
        out_specs=pl.BlockSpec((blk, LATDIM), lambda i: (i, 0)),
        out_shape=jax.ShapeDtypeStruct((h.shape[0], LATDIM), jnp.float32),
    )(h, W, b2)


def _pack_side(edge_index, edge_vals, src_offset):
    src = edge_index[1].astype(jnp.int32) + src_offset
    dst = edge_index[0].astype(jnp.int32)
    pad = NS * EDGES_PER_SUB - E_SIDE
    zpad = jnp.zeros((pad,), jnp.int32)
    src = jnp.concatenate([src, zpad]).reshape(NS, CHUNKS_PER_SUB, CHUNK)
    dst = jnp.concatenate([dst, zpad]).reshape(NS, CHUNKS_PER_SUB, CHUNK)
    vals = jnp.concatenate([edge_vals, zpad.astype(jnp.float32)])
    vals = vals.reshape(NS, CHUNKS_PER_SUB, CHUNK)
    return jnp.stack([src, dst], axis=2), vals


def kernel(uu_edge_index, uu_edge_vals, ii_edge_index, ii_edge_vals,
           uEmbeds, iEmbeds, W, b):
    pu, vu = _pack_side(uu_edge_index, uu_edge_vals, 0)
    pi, vi = _pack_side(ii_edge_index, ii_edge_vals, N_SIDE)
    packed = jnp.concatenate([pu[None], pi[None]]).reshape(-1)
    vals_all = jnp.concatenate([vu[None], vi[None]]).reshape(-1)
    emb_cat = jnp.concatenate([uEmbeds, iEmbeds], axis=0)

    h = _sc_spmm(packed, vals_all, emb_cat)
    return _tc_dense(h, W, b.reshape(1, LATDIM))


# speedup vs baseline: 5.1110x; 1.2194x over previous
import jax
import jax.numpy as jnp
from jax import lax
from jax.experimental import pallas as pl
from jax.experimental.pallas import tpu as pltpu
from jax.experimental.pallas import tpu_sc as plsc

N_SIDE = 10000
LATDIM = 128
E_SIDE = 320000
NC, NS, L = 2, 16, 16
CHUNK = 128
CHUNKS_PER_SUB = (-(-E_SIDE // (NS * CHUNK)) + 1) // 2 * 2
EDGES_PER_SUB = CHUNKS_PER_SUB * CHUNK
REC = 2 * CHUNK
EROWS = 80
NRCHUNK = N_SIDE // EROWS
NJJ = CHUNKS_PER_SUB // 2


def _sc_body(packed_hbm, vals_hbm, emb_hbm, out_hbm,
             buf_a, buf_b, vals_a, vals_b, rows0, rows1, accum,
             g0_sem, g1_sem, s0_sem, s1_sem, r_sem):
    c = lax.axis_index("c")
    s = lax.axis_index("s")
    zeros16 = jnp.zeros((L,), jnp.float32)

    nrc = (NRCHUNK - 1 - s) // NS + 1

    def _zrow(r, carry):
        for k in range(LATDIM // L):
            rows0[r, pl.ds(k * L, L)] = zeros16
        return carry
    lax.fori_loop(0, EROWS, _zrow, None)

    def _zchunk(t, carry):
        pltpu.sync_copy(rows0.at[pl.ds(0, EROWS)],
                        accum.at[pl.ds((s + t * NS) * EROWS, EROWS)])
        return carry
    lax.fori_loop(0, nrc, _zchunk, None)
    plsc.subcore_barrier()

    w = c * NS + s

    def _load_rec(j, buf, vls):
        base = w * CHUNKS_PER_SUB + j
        d1 = pltpu.async_copy(packed_hbm.at[pl.ds(base * REC, REC)], buf,
                              r_sem)
        d2 = pltpu.async_copy(vals_hbm.at[pl.ds(base * CHUNK, CHUNK)], vls,
                              r_sem)
        return d1, d2

    def _scale(rows, vls):
        @plsc.parallel_loop(0, CHUNK, 1, unroll=8)
        def _edge(e):
            g16 = (e // L) * L
            val16 = vls[pl.ds(g16, L)]
            bval = lax.gather(
                val16, jnp.full((L, 1), e - g16, jnp.int32),
                dimension_numbers=lax.GatherDimensionNumbers(
                    offset_dims=(), collapsed_slice_dims=(0,),
                    start_index_map=(0,)),
                slice_sizes=(1,),
                mode=lax.GatherScatterMode.PROMISE_IN_BOUNDS)
            for k in range(LATDIM // L):
                sl = pl.ds(k * L, L)
                rows[e, sl] = rows[e, sl] * bval

    def _fire_scatters(rows, buf, sem):
        for g in range(CHUNK // L):
            dvec = buf[pl.ds(CHUNK + g * L, L)]
            pltpu.async_copy(rows.at[pl.ds(g * L, L)], accum.at[dvec], sem,
                             add=True)

    def _drain_rows(rows, sem):
        pltpu.make_async_copy(emb_hbm.at[pl.ds(0, CHUNK)], rows, sem).wait()

    d1, d2 = _load_rec(0, buf_a, vals_a)
    d3, d4 = _load_rec(1, buf_b, vals_b)
    d1.wait(); d2.wait(); d3.wait(); d4.wait()
    pltpu.async_copy(emb_hbm.at[buf_a.at[pl.ds(0, CHUNK)]], rows0, g0_sem)

    def _body(jj, carry):
        j0 = 2 * jj

        @pl.when(jj > 0)
        def _():
            _drain_rows(rows1, s1_sem)
        pltpu.async_copy(emb_hbm.at[buf_b.at[pl.ds(0, CHUNK)]], rows1, g1_sem)

        pltpu.make_async_copy(emb_hbm.at[pl.ds(0, CHUNK)], rows0,
                              g0_sem).wait()
        _scale(rows0, vals_a)
        _fire_scatters(rows0, buf_a, s0_sem)

        @pl.when(jj < NJJ - 1)
        def _():
            ra1, ra2 = _load_rec(j0 + 2, buf_a, vals_a)

            pltpu.make_async_copy(emb_hbm.at[pl.ds(0, CHUNK)], rows1,
                                  g1_sem).wait()
            _scale(rows1, vals_b)
            _fire_scatters(rows1, buf_b, s1_sem)
            rb1, rb2 = _load_rec(j0 + 3, buf_b, vals_b)

            _drain_rows(rows0, s0_sem)
            ra1.wait(); ra2.wait(); rb1.wait(); rb2.wait()
            pltpu.async_copy(emb_hbm.at[buf_a.at[pl.ds(0, CHUNK)]], rows0,
                             g0_sem)

        @pl.when(jj == NJJ - 1)
        def _():
            pltpu.make_async_copy(emb_hbm.at[pl.ds(0, CHUNK)], rows1,
                                  g1_sem).wait()
            _scale(rows1, vals_b)
            _fire_scatters(rows1, buf_b, s1_sem)
            _drain_rows(rows0, s0_sem)
            _drain_rows(rows1, s1_sem)
        return carry
    lax.fori_loop(0, NJJ, _body, None)
    plsc.subcore_barrier()

    def _echunk(t, carry):
        r0 = (s + t * NS) * EROWS
        pltpu.sync_copy(accum.at[pl.ds(r0, EROWS)], rows0.at[pl.ds(0, EROWS)])
        pltpu.sync_copy(emb_hbm.at[pl.ds(c * N_SIDE + r0, EROWS)],
                        rows1.at[pl.ds(0, EROWS)])

        def _erow(r, cc):
            for k in range(LATDIM // L):
                sl = pl.ds(k * L, L)
                rows0[r, sl] = jnp.maximum(rows0[r, sl], 0.0) + rows1[r, sl]
            return cc
        lax.fori_loop(0, EROWS, _erow, None)
        pltpu.sync_copy(rows0.at[pl.ds(0, EROWS)],
                        out_hbm.at[pl.ds(c * N_SIDE + r0, EROWS)])
        return carry
    lax.fori_loop(0, nrc, _echunk, None)


_sc_spmm = pl.kernel(
    _sc_body,
    out_type=jax.ShapeDtypeStruct((2 * N_SIDE, LATDIM), jnp.float32),
    mesh=plsc.VectorSubcoreMesh(core_axis_name="c", subcore_axis_name="s",
                                num_cores=NC, num_subcores=NS),
    scratch_types=[
        pltpu.VMEM((REC,), jnp.int32),
        pltpu.VMEM((REC,), jnp.int32),
        pltpu.VMEM((CHUNK,), jnp.float32),
        pltpu.VMEM((CHUNK,), jnp.float32),
        pltpu.VMEM((CHUNK, LATDIM), jnp.float32),
        pltpu.VMEM((CHUNK, LATDIM), jnp.float32),
        pltpu.VMEM_SHARED((N_SIDE, LATDIM), jnp.float32),
        pltpu.SemaphoreType.DMA,
        pltpu.SemaphoreType.DMA,
        pltpu.SemaphoreType.DMA,
        pltpu.SemaphoreType.DMA,
        pltpu.SemaphoreType.DMA,
    ],
)


def _tc_dense_body(h_ref, w_ref, b_ref, o_ref):
    acc = jnp.dot(h_ref[...], w_ref[...], preferred_element_type=jnp.float32)
    o_ref[...] = jnp.maximum(2.0 * acc + b_ref[...], 0.0)


def _tc_dense(h, W, b2):
    blk = 1000
    n = h.shape[0] // blk
    return pl.pallas_call(
        _tc_dense_body,
        grid=(n,),
        in_specs=[
            pl.BlockSpec((blk, LATDIM), lambda i: (i, 0)),
            pl.BlockSpec((LATDIM, LATDIM), lambda i: (0, 0)),
            pl.BlockSpec((1, LATDIM), lambda i: (0, 0)),
        ],
        out_specs=pl.BlockSpec((blk, LATDIM), lambda i: (i, 0)),
        out_shape=jax.ShapeDtypeStruct((h.shape[0], LATDIM), jnp.float32),
    )(h, W, b2)


def _pack_side(edge_index, edge_vals, src_offset):
    src = edge_index[1].astype(jnp.int32) + src_offset
    dst = edge_index[0].astype(jnp.int32)
    pad = NS * EDGES_PER_SUB - E_SIDE
    zpad = jnp.zeros((pad,), jnp.int32)
    src = jnp.concatenate([src, zpad]).reshape(NS, CHUNKS_PER_SUB, CHUNK)
    dst = jnp.concatenate([dst, zpad]).reshape(NS, CHUNKS_PER_SUB, CHUNK)
    vals = jnp.concatenate([edge_vals, zpad.astype(jnp.float32)])
    vals = vals.reshape(NS, CHUNKS_PER_SUB, CHUNK)
    return jnp.stack([src, dst], axis=2), vals


def kernel(uu_edge_index, uu_edge_vals, ii_edge_index, ii_edge_vals,
           uEmbeds, iEmbeds, W, b):
    pu, vu = _pack_side(uu_edge_index, uu_edge_vals, 0)
    pi, vi = _pack_side(ii_edge_index, ii_edge_vals, N_SIDE)
    packed = jnp.concatenate([pu[None], pi[None]]).reshape(-1)
    vals_all = jnp.concatenate([vu[None], vi[None]]).reshape(-1)
    emb_cat = jnp.concatenate([uEmbeds, iEmbeds], axis=0)

    h = _sc_spmm(packed, vals_all, emb_cat)
    return _tc_dense(h, W, b.reshape(1, LATDIM))


# speedup vs baseline: 5.1115x; 1.0001x over previous
import jax
import jax.numpy as jnp
from jax import lax
from jax.experimental import pallas as pl
from jax.experimental.pallas import tpu as pltpu
from jax.experimental.pallas import tpu_sc as plsc

N_SIDE = 10000
LATDIM = 128
E_SIDE = 320000
NC, NS, L = 2, 16, 16
CHUNK = 128
CHUNKS_PER_SUB = (-(-E_SIDE // (NS * CHUNK)) + 1) // 2 * 2
EDGES_PER_SUB = CHUNKS_PER_SUB * CHUNK
REC = 2 * CHUNK
EROWS = 80
NRCHUNK = N_SIDE // EROWS
NJJ = CHUNKS_PER_SUB // 2


def _sc_body(packed_hbm, vals_hbm, emb_hbm, out_hbm,
             buf_a, buf_b, vals_a, vals_b, rows0, rows1, accum,
             g0_sem, g1_sem, s0_sem, s1_sem, r_sem):
    c = lax.axis_index("c")
    s = lax.axis_index("s")
    zeros16 = jnp.zeros((L,), jnp.float32)

    nrc = (NRCHUNK - 1 - s) // NS + 1

    def _zrow(r, carry):
        for k in range(LATDIM // L):
            rows0[r, pl.ds(k * L, L)] = zeros16
        return carry
    lax.fori_loop(0, EROWS, _zrow, None)

    def _zchunk(t, carry):
        pltpu.sync_copy(rows0.at[pl.ds(0, EROWS)],
                        accum.at[pl.ds((s + t * NS) * EROWS, EROWS)])
        return carry
    lax.fori_loop(0, nrc, _zchunk, None)
    plsc.subcore_barrier()

    w = c * NS + s

    def _load_rec(j, buf, vls):
        base = w * CHUNKS_PER_SUB + j
        d1 = pltpu.async_copy(packed_hbm.at[pl.ds(base * REC, REC)], buf,
                              r_sem)
        d2 = pltpu.async_copy(vals_hbm.at[pl.ds(base * CHUNK, CHUNK)], vls,
                              r_sem)
        return d1, d2

    def _scale(rows, vls):
        @plsc.parallel_loop(0, CHUNK, 1, unroll=8)
        def _edge(e):
            g16 = (e // L) * L
            val16 = vls[pl.ds(g16, L)]
            bval = lax.gather(
                val16, jnp.full((L, 1), e - g16, jnp.int32),
                dimension_numbers=lax.GatherDimensionNumbers(
                    offset_dims=(), collapsed_slice_dims=(0,),
                    start_index_map=(0,)),
                slice_sizes=(1,),
                mode=lax.GatherScatterMode.PROMISE_IN_BOUNDS)
            for k in range(LATDIM // L):
                sl = pl.ds(k * L, L)
                rows[e, sl] = rows[e, sl] * bval

    def _fire_scatters(rows, buf, sem):
        for g in range(CHUNK // L):
            dvec = buf[pl.ds(CHUNK + g * L, L)]
            pltpu.async_copy(rows.at[pl.ds(g * L, L)], accum.at[dvec], sem,
                             add=True)

    def _drain_rows(rows, sem):
        pltpu.make_async_copy(emb_hbm.at[pl.ds(0, CHUNK)], rows, sem).wait()

    d1, d2 = _load_rec(0, buf_a, vals_a)
    d3, d4 = _load_rec(1, buf_b, vals_b)
    d1.wait(); d2.wait(); d3.wait(); d4.wait()
    def _gather(buf, rows, sem):
        q4 = CHUNK // 4
        for q in range(4):
            pltpu.async_copy(emb_hbm.at[buf.at[pl.ds(q * q4, q4)]],
                             rows.at[pl.ds(q * q4, q4)], sem)

    _gather(buf_a, rows0, g0_sem)

    def _body(jj, carry):
        j0 = 2 * jj

        @pl.when(jj > 0)
        def _():
            _drain_rows(rows1, s1_sem)
        _gather(buf_b, rows1, g1_sem)

        pltpu.make_async_copy(emb_hbm.at[pl.ds(0, CHUNK)], rows0,
                              g0_sem).wait()
        _scale(rows0, vals_a)
        _fire_scatters(rows0, buf_a, s0_sem)

        @pl.when(jj < NJJ - 1)
        def _():
            ra1, ra2 = _load_rec(j0 + 2, buf_a, vals_a)

            pltpu.make_async_copy(emb_hbm.at[pl.ds(0, CHUNK)], rows1,
                                  g1_sem).wait()
            _scale(rows1, vals_b)
            _fire_scatters(rows1, buf_b, s1_sem)
            rb1, rb2 = _load_rec(j0 + 3, buf_b, vals_b)

            _drain_rows(rows0, s0_sem)
            ra1.wait(); ra2.wait(); rb1.wait(); rb2.wait()
            _gather(buf_a, rows0, g0_sem)

        @pl.when(jj == NJJ - 1)
        def _():
            pltpu.make_async_copy(emb_hbm.at[pl.ds(0, CHUNK)], rows1,
                                  g1_sem).wait()
            _scale(rows1, vals_b)
            _fire_scatters(rows1, buf_b, s1_sem)
            _drain_rows(rows0, s0_sem)
            _drain_rows(rows1, s1_sem)
        return carry
    lax.fori_loop(0, NJJ, _body, None)
    plsc.subcore_barrier()

    def _echunk(t, carry):
        r0 = (s + t * NS) * EROWS
        pltpu.sync_copy(accum.at[pl.ds(r0, EROWS)], rows0.at[pl.ds(0, EROWS)])
        pltpu.sync_copy(emb_hbm.at[pl.ds(c * N_SIDE + r0, EROWS)],
                        rows1.at[pl.ds(0, EROWS)])

        def _erow(r, cc):
            for k in range(LATDIM // L):
                sl = pl.ds(k * L, L)
                rows0[r, sl] = jnp.maximum(rows0[r, sl], 0.0) + rows1[r, sl]
            return cc
        lax.fori_loop(0, EROWS, _erow, None)
        pltpu.sync_copy(rows0.at[pl.ds(0, EROWS)],
                        out_hbm.at[pl.ds(c * N_SIDE + r0, EROWS)])
        return carry
    lax.fori_loop(0, nrc, _echunk, None)


_sc_spmm = pl.kernel(
    _sc_body,
    out_type=jax.ShapeDtypeStruct((2 * N_SIDE, LATDIM), jnp.float32),
    mesh=plsc.VectorSubcoreMesh(core_axis_name="c", subcore_axis_name="s",
                                num_cores=NC, num_subcores=NS),
    scratch_types=[
        pltpu.VMEM((REC,), jnp.int32),
        pltpu.VMEM((REC,), jnp.int32),
        pltpu.VMEM((CHUNK,), jnp.float32),
        pltpu.VMEM((CHUNK,), jnp.float32),
        pltpu.VMEM((CHUNK, LATDIM), jnp.float32),
        pltpu.VMEM((CHUNK, LATDIM), jnp.float32),
        pltpu.VMEM_SHARED((N_SIDE, LATDIM), jnp.float32),
        pltpu.SemaphoreType.DMA,
        pltpu.SemaphoreType.DMA,
        pltpu.SemaphoreType.DMA,
        pltpu.SemaphoreType.DMA,
        pltpu.SemaphoreType.DMA,
    ],
)


def _tc_dense_body(h_ref, w_ref, b_ref, o_ref):
    acc = jnp.dot(h_ref[...], w_ref[...], preferred_element_type=jnp.float32)
    o_ref[...] = jnp.maximum(2.0 * acc + b_ref[...], 0.0)


def _tc_dense(h, W, b2):
    blk = 1000
    n = h.shape[0] // blk
    return pl.pallas_call(
        _tc_dense_body,
        grid=(n,),
        in_specs=[
            pl.BlockSpec((blk, LATDIM), lambda i: (i, 0)),
            pl.BlockSpec((LATDIM, LATDIM), lambda i: (0, 0)),
            pl.BlockSpec((1, LATDIM), lambda i: (0, 0)),
        ],
        out_specs=pl.BlockSpec((blk, LATDIM), lambda i: (i, 0)),
        out_shape=jax.ShapeDtypeStruct((h.shape[0], LATDIM), jnp.float32),
    )(h, W, b2)


def _pack_side(edge_index, edge_vals, src_offset):
    src = edge_index[1].astype(jnp.int32) + src_offset
    dst = edge_index[0].astype(jnp.int32)
    pad = NS * EDGES_PER_SUB - E_SIDE
    zpad = jnp.zeros((pad,), jnp.int32)
    src = jnp.concatenate([src, zpad]).reshape(NS, CHUNKS_PER_SUB, CHUNK)
    dst = jnp.concatenate([dst, zpad]).reshape(NS, CHUNKS_PER_SUB, CHUNK)
    vals = jnp.concatenate([edge_vals, zpad.astype(jnp.float32)])
    vals = vals.reshape(NS, CHUNKS_PER_SUB, CHUNK)
    return jnp.stack([src, dst], axis=2), vals


def kernel(uu_edge_index, uu_edge_vals, ii_edge_index, ii_edge_vals,
           uEmbeds, iEmbeds, W, b):
    pu, vu = _pack_side(uu_edge_index, uu_edge_vals, 0)
    pi, vi = _pack_side(ii_edge_index, ii_edge_vals, N_SIDE)
    packed = jnp.concatenate([pu[None], pi[None]]).reshape(-1)
    vals_all = jnp.concatenate([vu[None], vi[None]]).reshape(-1)
    emb_cat = jnp.concatenate([uEmbeds, iEmbeds], axis=0)

    h = _sc_spmm(packed, vals_all, emb_cat)
    return _tc_dense(h, W, b.reshape(1, LATDIM))


# speedup vs baseline: 5.2641x; 1.0298x over previous
import jax
import jax.numpy as jnp
from jax import lax
from jax.experimental import pallas as pl
from jax.experimental.pallas import tpu as pltpu
from jax.experimental.pallas import tpu_sc as plsc

N_SIDE = 10000
LATDIM = 128
E_SIDE = 320000
NC, NS, L = 2, 16, 16
CHUNK = 128
CHUNKS_PER_SUB = (-(-E_SIDE // (NS * CHUNK)) + 1) // 2 * 2
EDGES_PER_SUB = CHUNKS_PER_SUB * CHUNK
REC = 2 * CHUNK
EROWS = 80
NRCHUNK = N_SIDE // EROWS
NJJ = CHUNKS_PER_SUB // 2


def _sc_body(packed_hbm, vals_hbm, emb_hbm, out_hbm,
             buf_a, buf_b, vals_a, vals_b, rows0, rows1, accum,
             g0_sem, g1_sem, s0_sem, s1_sem, r_sem):
    c = lax.axis_index("c")
    s = lax.axis_index("s")
    zeros16 = jnp.zeros((L,), jnp.float32)

    nrc = (NRCHUNK - 1 - s) // NS + 1

    def _zrow(r, carry):
        for k in range(LATDIM // L):
            rows0[r, pl.ds(k * L, L)] = zeros16
        return carry
    lax.fori_loop(0, EROWS, _zrow, None)

    def _zchunk(t, carry):
        pltpu.sync_copy(rows0.at[pl.ds(0, EROWS)],
                        accum.at[pl.ds((s + t * NS) * EROWS, EROWS)])
        return carry
    lax.fori_loop(0, nrc, _zchunk, None)
    plsc.subcore_barrier()

    w = c * NS + s

    def _load_rec(j, buf, vls):
        base = w * CHUNKS_PER_SUB + j
        d1 = pltpu.async_copy(packed_hbm.at[pl.ds(base * REC, REC)], buf,
                              r_sem)
        d2 = pltpu.async_copy(vals_hbm.at[pl.ds(base * CHUNK, CHUNK)], vls,
                              r_sem)
        return d1, d2

    def _scale(rows, vls):
        @plsc.parallel_loop(0, CHUNK, 1, unroll=8)
        def _edge(e):
            g16 = (e // L) * L
            val16 = vls[pl.ds(g16, L)]
            bval = lax.gather(
                val16, jnp.full((L, 1), e - g16, jnp.int32),
                dimension_numbers=lax.GatherDimensionNumbers(
                    offset_dims=(), collapsed_slice_dims=(0,),
                    start_index_map=(0,)),
                slice_sizes=(1,),
                mode=lax.GatherScatterMode.PROMISE_IN_BOUNDS)
            for k in range(LATDIM // L):
                sl = pl.ds(k * L, L)
                rows[e, sl] = rows[e, sl] * bval

    def _fire_scatters(rows, buf, sem):
        for g in range(CHUNK // L):
            dvec = buf[pl.ds(CHUNK + g * L, L)]
            pltpu.async_copy(rows.at[pl.ds(g * L, L)], accum.at[dvec], sem,
                             add=True)

    def _drain_rows(rows, sem):
        pltpu.make_async_copy(emb_hbm.at[pl.ds(0, CHUNK)], rows, sem).wait()

    d1, d2 = _load_rec(0, buf_a, vals_a)
    d3, d4 = _load_rec(1, buf_b, vals_b)
    d1.wait(); d2.wait(); d3.wait(); d4.wait()
    pltpu.async_copy(emb_hbm.at[buf_a.at[pl.ds(0, CHUNK)]], rows0, g0_sem)

    def _body(jj, carry):
        j0 = 2 * jj

        @pl.when(jj > 0)
        def _():
            _drain_rows(rows1, s1_sem)
        pltpu.async_copy(emb_hbm.at[buf_b.at[pl.ds(0, CHUNK)]], rows1, g1_sem)

        pltpu.make_async_copy(emb_hbm.at[pl.ds(0, CHUNK)], rows0,
                              g0_sem).wait()
        _scale(rows0, vals_a)
        _fire_scatters(rows0, buf_a, s0_sem)

        @pl.when(jj < NJJ - 1)
        def _():
            ra1, ra2 = _load_rec(j0 + 2, buf_a, vals_a)

            pltpu.make_async_copy(emb_hbm.at[pl.ds(0, CHUNK)], rows1,
                                  g1_sem).wait()
            _scale(rows1, vals_b)
            _fire_scatters(rows1, buf_b, s1_sem)
            rb1, rb2 = _load_rec(j0 + 3, buf_b, vals_b)

            _drain_rows(rows0, s0_sem)
            ra1.wait(); ra2.wait(); rb1.wait(); rb2.wait()
            pltpu.async_copy(emb_hbm.at[buf_a.at[pl.ds(0, CHUNK)]], rows0,
                             g0_sem)

        @pl.when(jj == NJJ - 1)
        def _():
            pltpu.make_async_copy(emb_hbm.at[pl.ds(0, CHUNK)], rows1,
                                  g1_sem).wait()
            _scale(rows1, vals_b)
            _fire_scatters(rows1, buf_b, s1_sem)
            _drain_rows(rows0, s0_sem)
            _drain_rows(rows1, s1_sem)
        return carry
    lax.fori_loop(0, NJJ, _body, None)
    plsc.subcore_barrier()

    def _echunk(t, carry):
        r0 = (s + t * NS) * EROWS
        pltpu.sync_copy(accum.at[pl.ds(r0, EROWS)],
                        out_hbm.at[pl.ds(c * N_SIDE + r0, EROWS)])
        return carry
    lax.fori_loop(0, nrc, _echunk, None)


_sc_spmm = pl.kernel(
    _sc_body,
    out_type=jax.ShapeDtypeStruct((2 * N_SIDE, LATDIM), jnp.float32),
    mesh=plsc.VectorSubcoreMesh(core_axis_name="c", subcore_axis_name="s",
                                num_cores=NC, num_subcores=NS),
    scratch_types=[
        pltpu.VMEM((REC,), jnp.int32),
        pltpu.VMEM((REC,), jnp.int32),
        pltpu.VMEM((CHUNK,), jnp.float32),
        pltpu.VMEM((CHUNK,), jnp.float32),
        pltpu.VMEM((CHUNK, LATDIM), jnp.float32),
        pltpu.VMEM((CHUNK, LATDIM), jnp.float32),
        pltpu.VMEM_SHARED((N_SIDE, LATDIM), jnp.float32),
        pltpu.SemaphoreType.DMA,
        pltpu.SemaphoreType.DMA,
        pltpu.SemaphoreType.DMA,
        pltpu.SemaphoreType.DMA,
        pltpu.SemaphoreType.DMA,
    ],
)


def _tc_dense_body(acc_ref, emb_ref, w_ref, b_ref, o_ref):
    h = jnp.maximum(acc_ref[...], 0.0) + emb_ref[...]
    acc = jnp.dot(h, w_ref[...], preferred_element_type=jnp.float32)
    o_ref[...] = jnp.maximum(2.0 * acc + b_ref[...], 0.0)


def _tc_dense(acc, emb, W, b2):
    blk = 1000
    n = acc.shape[0] // blk
    return pl.pallas_call(
        _tc_dense_body,
        grid=(n,),
        in_specs=[
            pl.BlockSpec((blk, LATDIM), lambda i: (i, 0)),
            pl.BlockSpec((blk, LATDIM), lambda i: (i, 0)),
            pl.BlockSpec((LATDIM, LATDIM), lambda i: (0, 0)),
            pl.BlockSpec((1, LATDIM), lambda i: (0, 0)),
        ],
        out_specs=pl.BlockSpec((blk, LATDIM), lambda i: (i, 0)),
        out_shape=jax.ShapeDtypeStruct((acc.shape[0], LATDIM), jnp.float32),
    )(acc, emb, W, b2)


def _pack_side(edge_index, edge_vals, src_offset):
    src = edge_index[1].astype(jnp.int32) + src_offset
    dst = edge_index[0].astype(jnp.int32)
    pad = NS * EDGES_PER_SUB - E_SIDE
    zpad = jnp.zeros((pad,), jnp.int32)
    src = jnp.concatenate([src, zpad]).reshape(NS, CHUNKS_PER_SUB, CHUNK)
    dst = jnp.concatenate([dst, zpad]).reshape(NS, CHUNKS_PER_SUB, CHUNK)
    vals = jnp.concatenate([edge_vals, zpad.astype(jnp.float32)])
    vals = vals.reshape(NS, CHUNKS_PER_SUB, CHUNK)
    return jnp.stack([src, dst], axis=2), vals


def kernel(uu_edge_index, uu_edge_vals, ii_edge_index, ii_edge_vals,
           uEmbeds, iEmbeds, W, b):
    pu, vu = _pack_side(uu_edge_index, uu_edge_vals, 0)
    pi, vi = _pack_side(ii_edge_index, ii_edge_vals, N_SIDE)
    packed = jnp.concatenate([pu[None], pi[None]]).reshape(-1)
    vals_all = jnp.concatenate([vu[None], vi[None]]).reshape(-1)
    emb_cat = jnp.concatenate([uEmbeds, iEmbeds], axis=0)

    acc = _sc_spmm(packed, vals_all, emb_cat)
    return _tc_dense(acc, emb_cat, W, b.reshape(1, LATDIM))
